# baseline (speedup 1.0000x reference)
import jax
import jax.numpy as jnp
from jax import lax
from jax.experimental import pallas as pl
from jax.experimental.pallas import tpu as pltpu
from jax.experimental.pallas import tpu_sc as plsc

N_NODES = 10000
D = 128
N_EDGES = 320000

NC = 2
NS = 16
CH = 128
SLAB = 16
SLABS = 10
CPT = SLAB * SLABS

N_PAD = 10240
ROWS_PER_TILE = N_PAD // NS
DUMMY_SLO = N_NODES + 112

E_ALL = 2 * N_EDGES
E_PAD = NC * NS * CPT * CH


def _sc_body(fea_hbm, idx_hbm, zeros_hbm, part_hbm,
             acc_sh, idx_v, rows_v, gsem, isem, ssem):
    c = lax.axis_index("c")
    s = lax.axis_index("s")
    w = c * NS + s
    chunk0 = w * CPT

    def load_slab_async(p, buf):
        pltpu.async_copy(idx_hbm.at[pl.ds(chunk0 + p * SLAB, SLAB)],
                         idx_v.at[buf], isem[buf])

    def wait_slab(buf):
        pltpu.make_async_copy(idx_hbm.at[pl.ds(chunk0, SLAB)],
                              idx_v.at[buf], isem[buf]).wait()

    def gather_async(sp, q, b):
        pltpu.async_copy(fea_hbm.at[idx_v.at[sp, q, 0]], rows_v.at[b],
                         gsem[b])

    def wait_gather(b):
        pltpu.make_async_copy(fea_hbm.at[idx_v.at[0, 0, 0]], rows_v.at[b],
                              gsem[b]).wait()

    def scatter_async(sp, q, b):
        pltpu.async_copy(rows_v.at[b], acc_sh.at[idx_v.at[sp, q, 1]],
                         ssem[b], add=True)

    def wait_scatter(b):
        pltpu.make_async_copy(rows_v.at[b], acc_sh.at[idx_v.at[0, 0, 1]],
                              ssem[b]).wait()

    load_slab_async(0, 0)
    load_slab_async(1, 1)
    row0 = s * ROWS_PER_TILE
    pltpu.sync_copy(zeros_hbm.at[pl.ds(row0, ROWS_PER_TILE)],
                    acc_sh.at[pl.ds(row0, ROWS_PER_TILE)])
    plsc.subcore_barrier()
    wait_slab(0)
    gather_async(0, 0, 0)

    def outer(P, carry):
        for sp in (0, 1):
            p = 2 * P + sp

            def inner(m, carry2):
                for b in (0, 1):
                    q = 2 * m + b
                    wait_gather(b)
                    scatter_async(sp, q, b)
                    if b == 0:
                        @pl.when((m > 0) | (p > 0))
                        def _():
                            wait_scatter(1)

                        @pl.when((m == 0) & (p >= 1) & (p < SLABS - 1))
                        def _():
                            load_slab_async(p + 1, 1 - sp)
                        gather_async(sp, q + 1, 1)
                    else:
                        wait_scatter(0)

                        @pl.when(m < SLAB // 2 - 1)
                        def _():
                            gather_async(sp, q + 1, 0)
                return carry2

            lax.fori_loop(0, SLAB // 2, inner, 0)

            @pl.when(p < SLABS - 1)
            def _():
                wait_slab(1 - sp)
                gather_async(1 - sp, 0, 0)
        return carry

    lax.fori_loop(0, SLABS // 2, outer, 0)
    wait_scatter(1)
    plsc.subcore_barrier()

    pltpu.sync_copy(acc_sh.at[pl.ds(row0, ROWS_PER_TILE)],
                    part_hbm.at[c, pl.ds(row0, ROWS_PER_TILE)])


def _sc_scatter(fea_pad, idx3d, zeros):
    mesh = plsc.VectorSubcoreMesh(core_axis_name="c", subcore_axis_name="s")
    return pl.kernel(
        _sc_body,
        out_type=jax.ShapeDtypeStruct((NC, N_PAD, D), jnp.float32),
        mesh=mesh,
        scratch_types=[
            pltpu.VMEM_SHARED((N_PAD, D), jnp.float32),
            pltpu.VMEM((2, SLAB, 2, CH), jnp.int32),
            pltpu.VMEM((2, CH, D), jnp.float32),
            [pltpu.SemaphoreType.DMA, pltpu.SemaphoreType.DMA],
            [pltpu.SemaphoreType.DMA, pltpu.SemaphoreType.DMA],
            [pltpu.SemaphoreType.DMA, pltpu.SemaphoreType.DMA],
        ],
    )(fea_pad, idx3d, zeros)


BLK = 2000


def _tc_self_body(fea_ref, ws_ref, bias_ref, out_ref):
    out_ref[...] = (
        jnp.dot(fea_ref[...], ws_ref[...], preferred_element_type=jnp.float32)
        + bias_ref[...]
    )


def _tc_self(fea, ws_t, bias):
    grid = (N_NODES // BLK,)
    return pl.pallas_call(
        _tc_self_body,
        grid=grid,
        in_specs=[
            pl.BlockSpec((BLK, D), lambda i: (i, 0)),
            pl.BlockSpec((D, D), lambda i: (0, 0)),
            pl.BlockSpec((1, D), lambda i: (0, 0)),
        ],
        out_specs=pl.BlockSpec((BLK, D), lambda i: (i, 0)),
        out_shape=jax.ShapeDtypeStruct((N_NODES, D), jnp.float32),
    )(fea, ws_t, bias)


def _tc_nei_body(self_ref, parts_ref, wn_ref, out_ref):
    nei = parts_ref[0] + parts_ref[1]
    out_ref[...] = self_ref[...] + jnp.dot(
        nei, wn_ref[...], preferred_element_type=jnp.float32)


def _tc_nei(self_out, parts, wn_t):
    grid = (N_NODES // BLK,)
    return pl.pallas_call(
        _tc_nei_body,
        grid=grid,
        in_specs=[
            pl.BlockSpec((BLK, D), lambda i: (i, 0)),
            pl.BlockSpec((NC, BLK, D), lambda i: (0, i, 0)),
            pl.BlockSpec((D, D), lambda i: (0, 0)),
        ],
        out_specs=pl.BlockSpec((BLK, D), lambda i: (i, 0)),
        out_shape=jax.ShapeDtypeStruct((N_NODES, D), jnp.float32),
    )(self_out, parts, wn_t)


def kernel(curr_fea, edge_index, W_self, b_self, W_neig, b_neig):
    src = edge_index[0].astype(jnp.int32)
    dst = edge_index[1].astype(jnp.int32)
    n_fill = E_PAD - E_ALL
    lane = jnp.arange(n_fill, dtype=jnp.int32) % 128
    gidx = jnp.concatenate([src, dst, lane]).reshape(-1, CH)
    sidx = jnp.concatenate([dst, src, DUMMY_SLO + lane]).reshape(-1, CH)
    idx3d = jnp.stack([gidx, sidx], axis=1)
    zeros = jnp.zeros((N_PAD, D), jnp.float32)

    parts = _sc_scatter(curr_fea, idx3d, zeros)

    bias = (b_self + b_neig).reshape(1, D)
    self_out = _tc_self(curr_fea, W_self.T, bias)
    return _tc_nei(self_out, parts, W_neig.T)


# speedup vs baseline: 1.2166x; 1.2166x over previous
import jax
import jax.numpy as jnp
from jax import lax
from jax.experimental import pallas as pl
from jax.experimental.pallas import tpu as pltpu
from jax.experimental.pallas import tpu_sc as plsc

N_NODES = 10000
D = 128
N_EDGES = 320000

NC = 2
NS = 16
CH = 112
SLAB = 6
SLABS = 30
CPT = SLAB * SLABS

N_PAD = 10112
ROWS_PER_TILE = N_PAD // NS
DUMMY_SLO = N_NODES

E_ALL = 2 * N_EDGES
E_PAD = NC * NS * CPT * CH


def _sc_body(fea_hbm, idx_hbm, zeros_hbm, part_hbm,
             acc_sh, idx_v, rows_v, gsem, isem, ssem):
    c = lax.axis_index("c")
    s = lax.axis_index("s")
    w = c * NS + s
    chunk0 = w * CPT

    def load_slab_async(p, buf):
        pltpu.async_copy(idx_hbm.at[pl.ds(chunk0 + p * SLAB, SLAB)],
                         idx_v.at[buf], isem[buf])

    def wait_slab(buf):
        pltpu.make_async_copy(idx_hbm.at[pl.ds(chunk0, SLAB)],
                              idx_v.at[buf], isem[buf]).wait()

    def gather_async(sp, q, b):
        pltpu.async_copy(fea_hbm.at[idx_v.at[sp, q, 0]], rows_v.at[b],
                         gsem[b])

    def wait_gather(b):
        pltpu.make_async_copy(fea_hbm.at[idx_v.at[0, 0, 0]], rows_v.at[b],
                              gsem[b]).wait()

    def scatter_async(sp, q, b):
        pltpu.async_copy(rows_v.at[b], acc_sh.at[idx_v.at[sp, q, 1]],
                         ssem[b], add=True)

    def wait_scatter(b):
        pltpu.make_async_copy(rows_v.at[b], acc_sh.at[idx_v.at[0, 0, 1]],
                              ssem[b]).wait()

    load_slab_async(0, 0)
    load_slab_async(1, 1)
    row0 = s * ROWS_PER_TILE
    pltpu.sync_copy(zeros_hbm.at[pl.ds(row0, ROWS_PER_TILE)],
                    acc_sh.at[pl.ds(row0, ROWS_PER_TILE)])
    plsc.subcore_barrier()
    wait_slab(0)
    gather_async(0, 0, 0)
    gather_async(0, 1, 1)

    def outer(P, carry):
        for sp in (0, 1):
            p = 2 * P + sp

            def inner(q, b, bn):
                wait_gather(b)
                scatter_async(sp, q, b)
                if q == 0:
                    @pl.when(p > 0)
                    def _():
                        wait_scatter(bn)

                    @pl.when((p >= 1) & (p < SLABS - 1))
                    def _():
                        load_slab_async(p + 1, 1 - sp)
                else:
                    wait_scatter(bn)
                if q < SLAB - 2:
                    gather_async(sp, q + 2, bn)
                elif q == SLAB - 2:
                    @pl.when(p < SLABS - 1)
                    def _():
                        wait_slab(1 - sp)
                        gather_async(1 - sp, 0, bn)
                else:
                    @pl.when(p < SLABS - 1)
                    def _():
                        gather_async(1 - sp, 1, bn)

            for q in range(SLAB):
                inner(q, q % 3, (q + 2) % 3)
        return carry

    lax.fori_loop(0, SLABS // 2, outer, 0)
    wait_scatter((SLAB * SLABS - 1) % 3)
    plsc.subcore_barrier()

    pltpu.sync_copy(acc_sh.at[pl.ds(row0, ROWS_PER_TILE)],
                    part_hbm.at[c, pl.ds(row0, ROWS_PER_TILE)])


def _sc_scatter(fea_pad, idx3d, zeros):
    mesh = plsc.VectorSubcoreMesh(core_axis_name="c", subcore_axis_name="s")
    return pl.kernel(
        _sc_body,
        out_type=jax.ShapeDtypeStruct((NC, N_PAD, D), jnp.float32),
        mesh=mesh,
        scratch_types=[
            pltpu.VMEM_SHARED((N_PAD, D), jnp.float32),
            pltpu.VMEM((2, SLAB, 2, CH), jnp.int32),
            pltpu.VMEM((3, CH, D), jnp.float32),
            [pltpu.SemaphoreType.DMA] * 3,
            [pltpu.SemaphoreType.DMA] * 2,
            [pltpu.SemaphoreType.DMA] * 3,
        ],
    )(fea_pad, idx3d, zeros)


BLK = 2000


def _tc_self_body(fea_ref, ws_ref, bias_ref, out_ref):
    out_ref[...] = (
        jnp.dot(fea_ref[...], ws_ref[...], preferred_element_type=jnp.float32)
        + bias_ref[...]
    )


def _tc_self(fea, ws_t, bias):
    grid = (N_NODES // BLK,)
    return pl.pallas_call(
        _tc_self_body,
        grid=grid,
        in_specs=[
            pl.BlockSpec((BLK, D), lambda i: (i, 0)),
            pl.BlockSpec((D, D), lambda i: (0, 0)),
            pl.BlockSpec((1, D), lambda i: (0, 0)),
        ],
        out_specs=pl.BlockSpec((BLK, D), lambda i: (i, 0)),
        out_shape=jax.ShapeDtypeStruct((N_NODES, D), jnp.float32),
    )(fea, ws_t, bias)


def _tc_nei_body(self_ref, parts_ref, wn_ref, out_ref):
    nei = parts_ref[0] + parts_ref[1]
    out_ref[...] = self_ref[...] + jnp.dot(
        nei, wn_ref[...], preferred_element_type=jnp.float32)


def _tc_nei(self_out, parts, wn_t):
    grid = (N_NODES // BLK,)
    return pl.pallas_call(
        _tc_nei_body,
        grid=grid,
        in_specs=[
            pl.BlockSpec((BLK, D), lambda i: (i, 0)),
            pl.BlockSpec((NC, BLK, D), lambda i: (0, i, 0)),
            pl.BlockSpec((D, D), lambda i: (0, 0)),
        ],
        out_specs=pl.BlockSpec((BLK, D), lambda i: (i, 0)),
        out_shape=jax.ShapeDtypeStruct((N_NODES, D), jnp.float32),
    )(self_out, parts, wn_t)


def kernel(curr_fea, edge_index, W_self, b_self, W_neig, b_neig):
    src = edge_index[0].astype(jnp.int32)
    dst = edge_index[1].astype(jnp.int32)
    n_fill = E_PAD - E_ALL
    lane = jnp.arange(n_fill, dtype=jnp.int32) % 112
    gidx = jnp.concatenate([src, dst, lane]).reshape(-1, CH)
    sidx = jnp.concatenate([dst, src, DUMMY_SLO + lane]).reshape(-1, CH)
    idx3d = jnp.stack([gidx, sidx], axis=1)
    zeros = jnp.zeros((N_PAD, D), jnp.float32)

    parts = _sc_scatter(curr_fea, idx3d, zeros)

    bias = (b_self + b_neig).reshape(1, D)
    self_out = _tc_self(curr_fea, W_self.T, bias)
    return _tc_nei(self_out, parts, W_neig.T)


# speedup vs baseline: 1.2219x; 1.0044x over previous
import jax
import jax.numpy as jnp
from jax import lax
from jax.experimental import pallas as pl
from jax.experimental.pallas import tpu as pltpu
from jax.experimental.pallas import tpu_sc as plsc

N_NODES = 10000
D = 128
N_EDGES = 320000

NC = 2
NS = 16
CH = 120
SLAB = 6
SLABS = 28
CPT = SLAB * SLABS

N_PAD = 10112
ROWS_PER_TILE = N_PAD // NS
DUMMY_SLO = N_NODES

E_ALL = 2 * N_EDGES
E_PAD = NC * NS * CPT * CH


def _sc_body(fea_hbm, idx_hbm, zeros_hbm, part_hbm,
             acc_sh, idx_v, rows_v, gsem, isem, ssem):
    c = lax.axis_index("c")
    s = lax.axis_index("s")
    w = c * NS + s
    chunk0 = w * CPT

    def load_slab_async(p, buf):
        pltpu.async_copy(idx_hbm.at[pl.ds(chunk0 + p * SLAB, SLAB)],
                         idx_v.at[buf], isem[buf])

    def wait_slab(buf):
        pltpu.make_async_copy(idx_hbm.at[pl.ds(chunk0, SLAB)],
                              idx_v.at[buf], isem[buf]).wait()

    def gather_async(sp, q, b):
        pltpu.async_copy(fea_hbm.at[idx_v.at[sp, q, 0]], rows_v.at[b],
                         gsem[b])

    def wait_gather(b):
        pltpu.make_async_copy(fea_hbm.at[idx_v.at[0, 0, 0]], rows_v.at[b],
                              gsem[b]).wait()

    def scatter_async(sp, q, b):
        pltpu.async_copy(rows_v.at[b], acc_sh.at[idx_v.at[sp, q, 1]],
                         ssem[b], add=True)

    def wait_scatter(b):
        pltpu.make_async_copy(rows_v.at[b], acc_sh.at[idx_v.at[0, 0, 1]],
                              ssem[b]).wait()

    load_slab_async(0, 0)
    load_slab_async(1, 1)
    row0 = s * ROWS_PER_TILE
    pltpu.sync_copy(zeros_hbm.at[pl.ds(row0, ROWS_PER_TILE)],
                    acc_sh.at[pl.ds(row0, ROWS_PER_TILE)])
    plsc.subcore_barrier()
    wait_slab(0)
    gather_async(0, 0, 0)
    gather_async(0, 1, 1)

    def outer(P, carry):
        for sp in (0, 1):
            p = 2 * P + sp

            def inner(q, b, bn):
                wait_gather(b)
                scatter_async(sp, q, b)
                if q == 0:
                    @pl.when(p > 0)
                    def _():
                        wait_scatter(bn)

                    @pl.when((p >= 1) & (p < SLABS - 1))
                    def _():
                        load_slab_async(p + 1, 1 - sp)
                else:
                    wait_scatter(bn)
                if q < SLAB - 2:
                    gather_async(sp, q + 2, bn)
                elif q == SLAB - 2:
                    @pl.when(p < SLABS - 1)
                    def _():
                        wait_slab(1 - sp)
                        gather_async(1 - sp, 0, bn)
                else:
                    @pl.when(p < SLABS - 1)
                    def _():
                        gather_async(1 - sp, 1, bn)

            for q in range(SLAB):
                inner(q, q % 3, (q + 2) % 3)
        return carry

    lax.fori_loop(0, SLABS // 2, outer, 0)
    wait_scatter((SLAB * SLABS - 1) % 3)
    plsc.subcore_barrier()

    pltpu.sync_copy(acc_sh.at[pl.ds(row0, ROWS_PER_TILE)],
                    part_hbm.at[c, pl.ds(row0, ROWS_PER_TILE)])


def _sc_scatter(fea_pad, idx3d, zeros):
    mesh = plsc.VectorSubcoreMesh(core_axis_name="c", subcore_axis_name="s")
    return pl.kernel(
        _sc_body,
        out_type=jax.ShapeDtypeStruct((NC, N_PAD, D), jnp.float32),
        mesh=mesh,
        scratch_types=[
            pltpu.VMEM_SHARED((N_PAD, D), jnp.float32),
            pltpu.VMEM((2, SLAB, 2, CH), jnp.int32),
            pltpu.VMEM((3, CH, D), jnp.float32),
            [pltpu.SemaphoreType.DMA] * 3,
            [pltpu.SemaphoreType.DMA] * 2,
            [pltpu.SemaphoreType.DMA] * 3,
        ],
    )(fea_pad, idx3d, zeros)


BLK = 2000


def _tc_self_body(fea_ref, ws_ref, bias_ref, out_ref):
    out_ref[...] = (
        jnp.dot(fea_ref[...], ws_ref[...], preferred_element_type=jnp.float32)
        + bias_ref[...]
    )


def _tc_self(fea, ws_t, bias):
    grid = (N_NODES // BLK,)
    return pl.pallas_call(
        _tc_self_body,
        grid=grid,
        in_specs=[
            pl.BlockSpec((BLK, D), lambda i: (i, 0)),
            pl.BlockSpec((D, D), lambda i: (0, 0)),
            pl.BlockSpec((1, D), lambda i: (0, 0)),
        ],
        out_specs=pl.BlockSpec((BLK, D), lambda i: (i, 0)),
        out_shape=jax.ShapeDtypeStruct((N_NODES, D), jnp.float32),
    )(fea, ws_t, bias)


def _tc_nei_body(self_ref, parts_ref, wn_ref, out_ref):
    nei = parts_ref[0] + parts_ref[1]
    out_ref[...] = self_ref[...] + jnp.dot(
        nei, wn_ref[...], preferred_element_type=jnp.float32)


def _tc_nei(self_out, parts, wn_t):
    grid = (N_NODES // BLK,)
    return pl.pallas_call(
        _tc_nei_body,
        grid=grid,
        in_specs=[
            pl.BlockSpec((BLK, D), lambda i: (i, 0)),
            pl.BlockSpec((NC, BLK, D), lambda i: (0, i, 0)),
            pl.BlockSpec((D, D), lambda i: (0, 0)),
        ],
        out_specs=pl.BlockSpec((BLK, D), lambda i: (i, 0)),
        out_shape=jax.ShapeDtypeStruct((N_NODES, D), jnp.float32),
    )(self_out, parts, wn_t)


def kernel(curr_fea, edge_index, W_self, b_self, W_neig, b_neig):
    src = edge_index[0].astype(jnp.int32)
    dst = edge_index[1].astype(jnp.int32)
    n_fill = E_PAD - E_ALL
    lane = jnp.arange(n_fill, dtype=jnp.int32) % 112
    gidx = jnp.concatenate([src, dst, lane]).reshape(-1, CH)
    sidx = jnp.concatenate([dst, src, DUMMY_SLO + lane]).reshape(-1, CH)
    idx3d = jnp.stack([gidx, sidx], axis=1)
    zeros = jnp.zeros((N_PAD, D), jnp.float32)

    parts = _sc_scatter(curr_fea, idx3d, zeros)

    bias = (b_self + b_neig).reshape(1, D)
    self_out = _tc_self(curr_fea, W_self.T, bias)
    return _tc_nei(self_out, parts, W_neig.T)
